# Initial kernel scaffold; baseline (speedup 1.0000x reference)
import functools

import jax
import jax.numpy as jnp
from jax import lax
from jax.experimental import pallas as pl
from jax.experimental.pallas import tpu as pltpu
from jax.experimental.pallas import tpu_sc as plsc

NC = 2
NS = 16
NW = NC * NS
B = 80


def _sc_segment_sum(t, src, dst, with_deg):
    n, d = t.shape
    e = src.shape[0]
    ew = e // NW
    nb = ew // B
    stripe = n // NS
    zrows = 125
    nz = stripe // zrows
    mesh = plsc.VectorSubcoreMesh(
        core_axis_name="c", subcore_axis_name="s",
        num_cores=NC, num_subcores=NS)

    out_type = [jax.ShapeDtypeStruct((NC, n, d), jnp.float32)]
    scratch = [
        pltpu.VMEM_SHARED((n, d), jnp.float32),
        pltpu.VMEM((B,), jnp.int32),
        pltpu.VMEM((B,), jnp.int32),
        pltpu.VMEM((B, d), jnp.float32),
        pltpu.VMEM((zrows, d), jnp.float32),
        pltpu.SemaphoreType.DMA,
    ]
    if with_deg:
        out_type.append(jax.ShapeDtypeStruct((NC, n, 16), jnp.float32))
        scratch += [
            pltpu.VMEM_SHARED((n, 16), jnp.float32),
            pltpu.VMEM((zrows, 16), jnp.float32),
            pltpu.VMEM((B, 16), jnp.float32),
        ]

    @functools.partial(pl.kernel, out_type=tuple(out_type), mesh=mesh,
                       scratch_types=scratch)
    def k(t_hbm, src_hbm, dst_hbm, *refs):
        if with_deg:
            (out_hbm, deg_hbm, acc, sidx, didx, rows, zbuf, sem,
             dacc, zdeg, ones) = refs
        else:
            out_hbm, acc, sidx, didx, rows, zbuf, sem = refs
        c = lax.axis_index("c")
        s = lax.axis_index("s")
        wid = s * NC + c

        def fill_z(i, _):
            for j in range(d // 16):
                zbuf[i, pl.ds(j * 16, 16)] = jnp.zeros((16,), jnp.float32)
            if with_deg:
                zdeg[i, :] = jnp.zeros((16,), jnp.float32)
            return 0
        lax.fori_loop(0, zrows, fill_z, 0)
        if with_deg:
            def fill_o(i, _):
                ones[i, :] = jnp.ones((16,), jnp.float32)
                return 0
            lax.fori_loop(0, B, fill_o, 0)

        r0 = s * stripe
        for kb in range(nz):
            pltpu.sync_copy(zbuf, acc.at[pl.ds(r0 + kb * zrows, zrows)])
            if with_deg:
                pltpu.sync_copy(zdeg, dacc.at[pl.ds(r0 + kb * zrows, zrows)])
        plsc.subcore_barrier()

        base0 = wid * ew

        def body(i, _):
            base = base0 + i * B
            pltpu.sync_copy(src_hbm.at[pl.ds(base, B)], sidx)
            pltpu.sync_copy(dst_hbm.at[pl.ds(base, B)], didx)
            pltpu.async_copy(t_hbm.at[sidx], rows, sem).wait()
            pltpu.sync_copy(rows, acc.at[didx], add=True)
            if with_deg:
                pltpu.sync_copy(ones, dacc.at[didx], add=True)
            return 0
        lax.fori_loop(0, nb, body, 0)
        plsc.subcore_barrier()

        pltpu.sync_copy(acc.at[pl.ds(r0, stripe)],
                        out_hbm.at[c, pl.ds(r0, stripe)])
        if with_deg:
            pltpu.sync_copy(dacc.at[pl.ds(r0, stripe)],
                            deg_hbm.at[c, pl.ds(r0, stripe)])

    return k(t, src, dst)


_ROWS = 2000


def _mm2(x, wa, wb):
    n, d = x.shape

    def body(x_ref, wa_ref, wb_ref, oa_ref, ob_ref):
        xb = x_ref[...]
        oa_ref[...] = jnp.dot(xb, wa_ref[...],
                              preferred_element_type=jnp.float32)
        ob_ref[...] = jnp.dot(xb, wb_ref[...],
                              preferred_element_type=jnp.float32)

    w_spec = pl.BlockSpec((d, d), lambda i: (0, 0))
    r_spec = pl.BlockSpec((_ROWS, d), lambda i: (i, 0))
    return pl.pallas_call(
        body,
        grid=(n // _ROWS,),
        in_specs=[r_spec, w_spec, w_spec],
        out_specs=[r_spec, r_spec],
        out_shape=[jax.ShapeDtypeStruct((n, d), jnp.float32)] * 2,
    )(x, wa, wb)


def _mid(s1, p1, deg, b1, w2n, w2s):
    n, d = s1.shape

    def body(s_ref, p_ref, d_ref, b_ref, wn_ref, ws_ref, t_ref, o_ref):
        neigh = p_ref[0] + p_ref[1]
        dg = d_ref[0, :, 0:1] + d_ref[1, :, 0:1]
        h = s_ref[...] + neigh / jnp.maximum(dg, 1.0) + b_ref[...]
        h = jnp.maximum(h, 0.0)
        t_ref[...] = jnp.dot(h, wn_ref[...],
                             preferred_element_type=jnp.float32)
        o_ref[...] = jnp.dot(h, ws_ref[...],
                             preferred_element_type=jnp.float32)

    r_spec = pl.BlockSpec((_ROWS, d), lambda i: (i, 0))
    w_spec = pl.BlockSpec((d, d), lambda i: (0, 0))
    return pl.pallas_call(
        body,
        grid=(n // _ROWS,),
        in_specs=[
            r_spec,
            pl.BlockSpec((2, _ROWS, d), lambda i: (0, i, 0)),
            pl.BlockSpec((2, _ROWS, 16), lambda i: (0, i, 0)),
            pl.BlockSpec((1, d), lambda i: (0, 0)),
            w_spec, w_spec,
        ],
        out_specs=[r_spec, r_spec],
        out_shape=[jax.ShapeDtypeStruct((n, d), jnp.float32)] * 2,
    )(s1, p1, deg, b1, w2n, w2s)


def _final(s2, p2, deg, b2):
    n, d = s2.shape

    def body(s_ref, p_ref, d_ref, b_ref, o_ref):
        neigh = p_ref[0] + p_ref[1]
        dg = d_ref[0, :, 0:1] + d_ref[1, :, 0:1]
        o_ref[...] = s_ref[...] + neigh / jnp.maximum(dg, 1.0) + b_ref[...]

    r_spec = pl.BlockSpec((_ROWS, d), lambda i: (i, 0))
    return pl.pallas_call(
        body,
        grid=(n // _ROWS,),
        in_specs=[
            r_spec,
            pl.BlockSpec((2, _ROWS, d), lambda i: (0, i, 0)),
            pl.BlockSpec((2, _ROWS, 16), lambda i: (0, i, 0)),
            pl.BlockSpec((1, d), lambda i: (0, 0)),
        ],
        out_specs=r_spec,
        out_shape=jax.ShapeDtypeStruct((n, d), jnp.float32),
    )(s2, p2, deg, b2)


def kernel(x, edge_index, W1_self, W1_neigh, b1, W2_self, W2_neigh, b2):
    src = edge_index[0]
    dst = edge_index[1]
    d = x.shape[1]
    t1, s1 = _mm2(x, W1_neigh, W1_self)
    p1, deg = _sc_segment_sum(t1, src, dst, with_deg=True)
    t2, s2 = _mid(s1, p1, deg, b1.reshape(1, d), W2_neigh, W2_self)
    (p2,) = _sc_segment_sum(t2, src, dst, with_deg=False)
    return _final(s2, p2, deg, b2.reshape(1, d))


# speedup vs baseline: 4.3967x; 4.3967x over previous
import functools

import jax
import jax.numpy as jnp
from jax import lax
from jax.experimental import pallas as pl
from jax.experimental.pallas import tpu as pltpu
from jax.experimental.pallas import tpu_sc as plsc

NC = 2
NS = 16
NW = NC * NS
B = 80


def _sc_segment_sum(t, src, dst, counts_only):
    n, d = t.shape
    e = src.shape[0]
    ew = e // NW
    nb = ew // B
    nzblk = n // B
    zper = -(-nzblk // NS)
    mesh = plsc.VectorSubcoreMesh(
        core_axis_name="c", subcore_axis_name="s",
        num_cores=NC, num_subcores=NS)

    out_type = jax.ShapeDtypeStruct((NC * n, d), jnp.float32)
    scratch = [
        pltpu.VMEM_SHARED((n, d), jnp.float32),
        pltpu.VMEM((B,), jnp.int32),
        pltpu.VMEM((B,), jnp.int32),
        pltpu.VMEM((B, d), jnp.float32),
        pltpu.SemaphoreType.DMA,
    ]

    @functools.partial(pl.kernel, out_type=out_type, mesh=mesh,
                       scratch_types=scratch)
    def k(t_hbm, src_hbm, dst_hbm, out_hbm, acc, sidx, didx, rows, sem):
        c = lax.axis_index("c")
        s = lax.axis_index("s")
        wid = s * NC + c
        zper = -(-nzblk // NS)

        def fill_rows(val):
            def fr(i, _):
                for j in range(d // 16):
                    rows[i, pl.ds(j * 16, 16)] = jnp.full((16,), val,
                                                          jnp.float32)
                return 0
            lax.fori_loop(0, B, fr, 0)
        fill_rows(0.0)

        def fill_didx(base):
            def fi(j, _):
                didx[pl.ds(j * 16, 16)] = (lax.iota(jnp.int32, 16)
                                           + base + j * 16)
                return 0
            lax.fori_loop(0, B // 16, fi, 0)

        for k in range(zper):
            blk = lax.rem(s + NS * k, nzblk)
            fill_didx(blk * B)
            pltpu.sync_copy(rows, acc.at[didx])
        plsc.subcore_barrier()

        if counts_only:
            fill_rows(1.0)

        base0 = wid * ew

        def body(i, _):
            base = base0 + i * B
            pltpu.sync_copy(dst_hbm.at[pl.ds(base, B)], didx)
            if not counts_only:
                pltpu.sync_copy(src_hbm.at[pl.ds(base, B)], sidx)
                pltpu.async_copy(t_hbm.at[sidx], rows, sem).wait()
            pltpu.sync_copy(rows, acc.at[didx], add=True)
            return 0
        lax.fori_loop(0, nb, body, 0)
        plsc.subcore_barrier()

        for k in range(zper):
            blk = lax.rem(s + NS * k, nzblk)
            fill_didx(blk * B)
            pltpu.async_copy(acc.at[didx], rows, sem).wait()
            pltpu.sync_copy(rows, out_hbm.at[pl.ds(c * n + blk * B, B)])

    return k(t, src, dst)


_ROWS = 2000


def _mm2(x, wa, wb):
    n, d = x.shape

    def body(x_ref, wa_ref, wb_ref, oa_ref, ob_ref):
        xb = x_ref[...]
        oa_ref[...] = jnp.dot(xb, wa_ref[...],
                              preferred_element_type=jnp.float32)
        ob_ref[...] = jnp.dot(xb, wb_ref[...],
                              preferred_element_type=jnp.float32)

    w_spec = pl.BlockSpec((d, d), lambda i: (0, 0))
    r_spec = pl.BlockSpec((_ROWS, d), lambda i: (i, 0))
    return pl.pallas_call(
        body,
        grid=(n // _ROWS,),
        in_specs=[r_spec, w_spec, w_spec],
        out_specs=[r_spec, r_spec],
        out_shape=[jax.ShapeDtypeStruct((n, d), jnp.float32)] * 2,
    )(x, wa, wb)


def _mid(s1, p1, deg, b1, w2n, w2s):
    n, d = s1.shape

    def body(s_ref, p_ref, d_ref, b_ref, wn_ref, ws_ref, t_ref, o_ref):
        neigh = p_ref[0] + p_ref[1]
        dg = d_ref[0, :, 0:1] + d_ref[1, :, 0:1]
        h = s_ref[...] + neigh / jnp.maximum(dg, 1.0) + b_ref[...]
        h = jnp.maximum(h, 0.0)
        t_ref[...] = jnp.dot(h, wn_ref[...],
                             preferred_element_type=jnp.float32)
        o_ref[...] = jnp.dot(h, ws_ref[...],
                             preferred_element_type=jnp.float32)

    r_spec = pl.BlockSpec((_ROWS, d), lambda i: (i, 0))
    w_spec = pl.BlockSpec((d, d), lambda i: (0, 0))
    return pl.pallas_call(
        body,
        grid=(n // _ROWS,),
        in_specs=[
            r_spec,
            pl.BlockSpec((2, _ROWS, d), lambda i: (0, i, 0)),
            pl.BlockSpec((2, _ROWS, d), lambda i: (0, i, 0)),
            pl.BlockSpec((1, d), lambda i: (0, 0)),
            w_spec, w_spec,
        ],
        out_specs=[r_spec, r_spec],
        out_shape=[jax.ShapeDtypeStruct((n, d), jnp.float32)] * 2,
    )(s1, p1, deg, b1, w2n, w2s)


def _final(s2, p2, deg, b2):
    n, d = s2.shape

    def body(s_ref, p_ref, d_ref, b_ref, o_ref):
        neigh = p_ref[0] + p_ref[1]
        dg = d_ref[0, :, 0:1] + d_ref[1, :, 0:1]
        o_ref[...] = s_ref[...] + neigh / jnp.maximum(dg, 1.0) + b_ref[...]

    r_spec = pl.BlockSpec((_ROWS, d), lambda i: (i, 0))
    return pl.pallas_call(
        body,
        grid=(n // _ROWS,),
        in_specs=[
            r_spec,
            pl.BlockSpec((2, _ROWS, d), lambda i: (0, i, 0)),
            pl.BlockSpec((2, _ROWS, d), lambda i: (0, i, 0)),
            pl.BlockSpec((1, d), lambda i: (0, 0)),
        ],
        out_specs=r_spec,
        out_shape=jax.ShapeDtypeStruct((n, d), jnp.float32),
    )(s2, p2, deg, b2)


def kernel(x, edge_index, W1_self, W1_neigh, b1, W2_self, W2_neigh, b2):
    src = edge_index[0]
    dst = edge_index[1]
    d = x.shape[1]
    n = x.shape[0]
    t1, s1 = _mm2(x, W1_neigh, W1_self)
    p1 = _sc_segment_sum(t1, src, dst, counts_only=False).reshape(NC, n, d)
    deg = _sc_segment_sum(t1, src, dst, counts_only=True).reshape(NC, n, d)
    t2, s2 = _mid(s1, p1, deg, b1.reshape(1, d), W2_neigh, W2_self)
    p2 = _sc_segment_sum(t2, src, dst, counts_only=False).reshape(NC, n, d)
    return _final(s2, p2, deg, b2.reshape(1, d))
